# Initial kernel scaffold; baseline (speedup 1.0000x reference)
import jax
import jax.numpy as jnp
from jax.experimental import pallas as pl


def kernel(z, codebook):
    raise NotImplementedError("write your pallas kernel here")


# speedup vs baseline: 1.1537x; 1.1537x over previous
import functools

import jax
import jax.numpy as jnp
from jax import lax
from jax.experimental import pallas as pl
from jax.experimental.pallas import tpu as pltpu
from jax.experimental.pallas import tpu_sc as plsc

N = 8192
E = 256
BETA = 0.25
BM = 256
NI = N // BM
NW = 32
RPW = N // NW
CH = 128
NCH = RPW // CH


def _normalize_rows(x):
    n = jnp.linalg.norm(x, ord=2, axis=-1, keepdims=True)
    return x / jnp.maximum(n, 1e-12)


def _k1_body(zb_ref, cnt_ref, nz_ref, nc_ref, idx_ref, loss_ref, acc_ref):
    i = pl.program_id(0)
    s = lax.dot_general(zb_ref[...], cnt_ref[...], (((1,), (0,)), ((), ())),
                        preferred_element_type=jnp.float32)
    H = N // 2
    s0, s1 = s[:, :H], s[:, H:]
    m0 = jnp.max(s0, axis=1, keepdims=True)
    m1 = jnp.max(s1, axis=1, keepdims=True)
    cols = lax.broadcasted_iota(jnp.int32, (BM, H), 1)
    li0 = jnp.min(jnp.where(s0 == m0, cols, H), axis=1, keepdims=True)
    li1 = jnp.min(jnp.where(s1 == m1, cols, H), axis=1, keepdims=True)
    use1 = m1 > m0.astype(jnp.bfloat16).astype(jnp.float32)
    li = jnp.where(use1, li1 + H, li0)
    m = jnp.where(use1, m1, m0)
    idx_ref[...] = li[:, 0]
    ncb = nc_ref[...][None, :]
    ncsel0 = jnp.sum(jnp.where(cols == li0, ncb[:, :H], 0.0), axis=1,
                     keepdims=True)
    ncsel1 = jnp.sum(jnp.where(cols == li1, ncb[:, H:], 0.0), axis=1,
                     keepdims=True)
    ncsel = jnp.where(use1, ncsel1, ncsel0)
    nzc = nz_ref[...].reshape(BM, 1)
    contrib = jnp.sum(nzc * nzc - 2.0 * m * nzc * ncsel + ncsel * ncsel)
    total = jnp.where(i == 0, 0.0, acc_ref[0]) + contrib
    acc_ref[0] = total

    @pl.when(i == NI - 1)
    def _():
        loss_ref[...] = jnp.full((8, 128), total * ((1.0 + BETA) / (N * E)),
                                 jnp.float32)


def _k1(z16, cnt16, nz, nc):
    return pl.pallas_call(
        _k1_body,
        grid=(NI,),
        in_specs=[
            pl.BlockSpec((BM, E), lambda i: (i, 0)),
            pl.BlockSpec((E, N), lambda i: (0, 0)),
            pl.BlockSpec((BM,), lambda i: (i,)),
            pl.BlockSpec((N,), lambda i: (0,)),
        ],
        out_specs=[
            pl.BlockSpec((BM,), lambda i: (i,)),
            pl.BlockSpec((8, 128), lambda i: (0, 0)),
        ],
        out_shape=[
            jax.ShapeDtypeStruct((N,), jnp.int32),
            jax.ShapeDtypeStruct((8, 128), jnp.float32),
        ],
        scratch_shapes=[pltpu.SMEM((1,), jnp.float32)],
    )(z16, cnt16, nz, nc)


def _k2_body(idx_ref, destT_ref, labelT_ref):
    idx8 = idx_ref[...]
    rio = lax.broadcasted_iota(jnp.int32, (128, 1), 0)
    cio = lax.broadcasted_iota(jnp.int32, (1, 64), 1)

    def body(c, acc):
        codes = c * 128 + rio
        red = jnp.zeros((128, 1), jnp.float32)
        for t in range(8):
            eq = codes == idx8[t:t + 1, :]
            red = jnp.maximum(
                red, jnp.max(jnp.where(eq, 1.0, 0.0), axis=1, keepdims=True))
        return acc + red * (cio == c).astype(jnp.float32)

    usedT = lax.fori_loop(0, 64, body, jnp.zeros((128, 64), jnp.float32))

    r128 = lax.broadcasted_iota(jnp.int32, (128, 128), 0)
    c128 = lax.broadcasted_iota(jnp.int32, (128, 128), 1)
    lower_incl = (c128 <= r128).astype(jnp.float32)
    inclT = lax.dot_general(lower_incl, usedT, (((1,), (0,)), ((), ())),
                            preferred_element_type=jnp.float32)
    coltot = inclT[127:128, :]
    a64 = lax.broadcasted_iota(jnp.int32, (64, 64), 0)
    b64 = lax.broadcasted_iota(jnp.int32, (64, 64), 1)
    upper_strict = (a64 < b64).astype(jnp.float32)
    coloff = lax.dot_general(coltot, upper_strict, (((1,), (0,)), ((), ())),
                             preferred_element_type=jnp.float32)
    csumT = inclT + coloff
    nu = jnp.max(csumT)
    flatj = (rio + 128 * cio).astype(jnp.float32)
    destT_ref[...] = jnp.where(usedT > 0.5, csumT - 1.0,
                               nu + flatj - csumT).astype(jnp.int32)
    labelT_ref[...] = (flatj < nu).astype(jnp.float32)


def _k2(idx8):
    return pl.pallas_call(
        _k2_body,
        in_specs=[pl.BlockSpec((8, N // 8), lambda: (0, 0))],
        out_specs=[
            pl.BlockSpec((128, 64), lambda: (0, 0)),
            pl.BlockSpec((128, 64), lambda: (0, 0)),
        ],
        out_shape=[
            jax.ShapeDtypeStruct((128, 64), jnp.int32),
            jax.ShapeDtypeStruct((128, 64), jnp.float32),
        ],
    )(idx8)


def _k3(codebook, dest3):
    mesh = plsc.VectorSubcoreMesh(core_axis_name="c", subcore_axis_name="s")

    @functools.partial(
        pl.kernel,
        out_type=jax.ShapeDtypeStruct((N, E), jnp.float32),
        mesh=mesh,
        scratch_types=[
            pltpu.VMEM((NCH, CH), jnp.int32),
            pltpu.VMEM((CH, E), jnp.float32),
        ],
    )
    def k(cb_hbm, d_hbm, out_hbm, idx_v, rows_v):
        cid = lax.axis_index("c")
        sid = lax.axis_index("s")
        wid = sid * 2 + cid
        pltpu.sync_copy(d_hbm.at[wid], idx_v)
        for kk in range(NCH):
            base = wid * RPW + kk * CH
            pltpu.sync_copy(cb_hbm.at[pl.ds(base, CH)], rows_v)
            pltpu.sync_copy(rows_v, out_hbm.at[idx_v.at[kk]])

    return k(codebook, dest3)


def kernel(z, codebook):
    zn = _normalize_rows(z)
    cn = _normalize_rows(codebook)
    z16, cnt16 = lax.optimization_barrier(
        (zn.astype(jnp.bfloat16), cn.T.astype(jnp.bfloat16)))
    nz = jnp.linalg.norm(z, ord=2, axis=-1)
    nc = jnp.linalg.norm(codebook, ord=2, axis=-1)
    idx, lossbuf = _k1(z16, cnt16, nz, nc)
    loss = lossbuf[0, 0]
    destT, labelT = _k2(idx.reshape(8, N // 8))
    dest = destT.T.reshape(-1)
    label = labelT.T.reshape(-1)
    emb = _k3(codebook, dest.reshape(NW, NCH, CH))
    return emb, label, loss


# speedup vs baseline: 1.1920x; 1.0332x over previous
import functools

import jax
import jax.numpy as jnp
from jax import lax
from jax.experimental import pallas as pl
from jax.experimental.pallas import tpu as pltpu
from jax.experimental.pallas import tpu_sc as plsc

N = 8192
E = 256
BETA = 0.25
BM = 256
NI = N // BM
NW = 32
RPW = N // NW
CH = 128
NCH = RPW // CH


def _normalize_rows(x):
    n = jnp.linalg.norm(x, ord=2, axis=-1, keepdims=True)
    return x / jnp.maximum(n, 1e-12)


def _k1_body(zb_ref, cn_ref, nz_ref, nc_ref, idx_ref, loss_ref, acc_ref):
    i = pl.program_id(0)
    s = lax.dot_general(zb_ref[...], cn_ref[...], (((1,), (1,)), ((), ())),
                        preferred_element_type=jnp.float32)
    H = N // 2
    s0, s1 = s[:, :H], s[:, H:]
    m0 = jnp.max(s0, axis=1, keepdims=True)
    m1 = jnp.max(s1, axis=1, keepdims=True)
    cols = lax.broadcasted_iota(jnp.int32, (BM, H), 1)
    li0 = jnp.min(jnp.where(s0 == m0, cols, H), axis=1, keepdims=True)
    li1 = jnp.min(jnp.where(s1 == m1, cols, H), axis=1, keepdims=True)
    use1 = m1 > m0.astype(jnp.bfloat16).astype(jnp.float32)
    li = jnp.where(use1, li1 + H, li0)
    m = jnp.where(use1, m1, m0)
    idx_ref[...] = li[:, 0]
    ncb = nc_ref[...][None, :]
    ncsel0 = jnp.sum(jnp.where(cols == li0, ncb[:, :H], 0.0), axis=1,
                     keepdims=True)
    ncsel1 = jnp.sum(jnp.where(cols == li1, ncb[:, H:], 0.0), axis=1,
                     keepdims=True)
    ncsel = jnp.where(use1, ncsel1, ncsel0)
    nzc = nz_ref[...].reshape(BM, 1)
    contrib = jnp.sum(nzc * nzc - 2.0 * m * nzc * ncsel + ncsel * ncsel)
    total = jnp.where(i == 0, 0.0, acc_ref[0]) + contrib
    acc_ref[0] = total

    @pl.when(i == NI - 1)
    def _():
        loss_ref[...] = jnp.full((8, 128), total * ((1.0 + BETA) / (N * E)),
                                 jnp.float32)


def _k1(z16, cn16, nz, nc):
    return pl.pallas_call(
        _k1_body,
        grid=(NI,),
        in_specs=[
            pl.BlockSpec((BM, E), lambda i: (i, 0)),
            pl.BlockSpec((N, E), lambda i: (0, 0)),
            pl.BlockSpec((BM,), lambda i: (i,)),
            pl.BlockSpec((N,), lambda i: (0,)),
        ],
        out_specs=[
            pl.BlockSpec((BM,), lambda i: (i,)),
            pl.BlockSpec((8, 128), lambda i: (0, 0)),
        ],
        out_shape=[
            jax.ShapeDtypeStruct((N,), jnp.int32),
            jax.ShapeDtypeStruct((8, 128), jnp.float32),
        ],
        scratch_shapes=[pltpu.SMEM((1,), jnp.float32)],
    )(z16, cn16, nz, nc)


def _k2_body(idx_ref, dest_ref, label_ref):
    idx8 = idx_ref[...]
    rio = lax.broadcasted_iota(jnp.int32, (128, 1), 0)
    cio = lax.broadcasted_iota(jnp.int32, (1, 64), 1)

    def body(c, acc):
        codes = c * 128 + rio
        red = jnp.zeros((128, 1), jnp.float32)
        for t in range(8):
            eq = codes == idx8[t:t + 1, :]
            red = jnp.maximum(
                red, jnp.max(jnp.where(eq, 1.0, 0.0), axis=1, keepdims=True))
        return acc + red * (cio == c).astype(jnp.float32)

    usedT = lax.fori_loop(0, 64, body, jnp.zeros((128, 64), jnp.float32))

    r128 = lax.broadcasted_iota(jnp.int32, (128, 128), 0)
    c128 = lax.broadcasted_iota(jnp.int32, (128, 128), 1)
    lower_incl = (c128 <= r128).astype(jnp.float32)
    inclT = lax.dot_general(lower_incl, usedT, (((1,), (0,)), ((), ())),
                            preferred_element_type=jnp.float32)
    coltot = inclT[127:128, :]
    a64 = lax.broadcasted_iota(jnp.int32, (64, 64), 0)
    b64 = lax.broadcasted_iota(jnp.int32, (64, 64), 1)
    upper_strict = (a64 < b64).astype(jnp.float32)
    coloff = lax.dot_general(coltot, upper_strict, (((1,), (0,)), ((), ())),
                             preferred_element_type=jnp.float32)
    csumT = inclT + coloff
    nu = jnp.max(csumT)
    flatj = (rio + 128 * cio).astype(jnp.float32)
    destT = jnp.where(usedT > 0.5, csumT - 1.0,
                      nu + flatj - csumT).astype(jnp.int32)
    labelT = (flatj < nu).astype(jnp.float32)
    dest_ref[...] = destT.T
    label_ref[...] = labelT.T


def _k2(idx8):
    return pl.pallas_call(
        _k2_body,
        in_specs=[pl.BlockSpec((8, N // 8), lambda: (0, 0))],
        out_specs=[
            pl.BlockSpec((64, 128), lambda: (0, 0)),
            pl.BlockSpec((64, 128), lambda: (0, 0)),
        ],
        out_shape=[
            jax.ShapeDtypeStruct((64, 128), jnp.int32),
            jax.ShapeDtypeStruct((64, 128), jnp.float32),
        ],
    )(idx8)


def _k3(codebook, dest3):
    mesh = plsc.VectorSubcoreMesh(core_axis_name="c", subcore_axis_name="s")

    @functools.partial(
        pl.kernel,
        out_type=jax.ShapeDtypeStruct((N, E), jnp.float32),
        mesh=mesh,
        scratch_types=[
            pltpu.VMEM((NCH, CH), jnp.int32),
            pltpu.VMEM((CH, E), jnp.float32),
        ],
    )
    def k(cb_hbm, d_hbm, out_hbm, idx_v, rows_v):
        cid = lax.axis_index("c")
        sid = lax.axis_index("s")
        wid = sid * 2 + cid
        pltpu.sync_copy(d_hbm.at[wid], idx_v)
        for kk in range(NCH):
            base = wid * RPW + kk * CH
            pltpu.sync_copy(cb_hbm.at[pl.ds(base, CH)], rows_v)
            pltpu.sync_copy(rows_v, out_hbm.at[idx_v.at[kk]])

    return k(codebook, dest3)


def kernel(z, codebook):
    zn = _normalize_rows(z)
    cn = _normalize_rows(codebook)
    z16, cn16 = lax.optimization_barrier(
        (zn.astype(jnp.bfloat16), cn.astype(jnp.bfloat16)))
    nz = jnp.linalg.norm(z, ord=2, axis=-1)
    nc = jnp.linalg.norm(codebook, ord=2, axis=-1)
    idx, lossbuf = _k1(z16, cn16, nz, nc)
    loss = lossbuf[0, 0]
    dest, label = _k2(idx.reshape(8, N // 8))
    emb = _k3(codebook, dest.reshape(NW, NCH, CH))
    return emb, label.reshape(-1), loss


# speedup vs baseline: 1.5956x; 1.3386x over previous
import dataclasses
import functools

import jax
import jax.numpy as jnp
from jax import lax
from jax.experimental import pallas as pl
from jax.experimental.pallas import tpu as pltpu
from jax.experimental.pallas import tpu_sc as plsc

N = 8192
E = 256
BETA = 0.25
BM = 256
NI = N // BM
NW = 32
RPW = N // NW
CH = 128
NCH = RPW // CH
L = 16


def _normalize_rows(x):
    n = jnp.linalg.norm(x, ord=2, axis=-1, keepdims=True)
    return x / jnp.maximum(n, 1e-12)


def _k1_body(zb_ref, cn_ref, idx_ref, m_ref):
    s = lax.dot_general(zb_ref[...], cn_ref[...], (((1,), (1,)), ((), ())),
                        preferred_element_type=jnp.float32)
    H = N // 2
    s0, s1 = s[:, :H], s[:, H:]
    m0 = jnp.max(s0, axis=1, keepdims=True)
    m1 = jnp.max(s1, axis=1, keepdims=True)
    cols = lax.broadcasted_iota(jnp.int32, (BM, H), 1)
    li0 = jnp.min(jnp.where(s0 == m0, cols, H), axis=1, keepdims=True)
    li1 = jnp.min(jnp.where(s1 == m1, cols, H), axis=1, keepdims=True)
    use1 = m1 > m0.astype(jnp.bfloat16).astype(jnp.float32)
    li = jnp.where(use1, li1 + H, li0)
    m = jnp.where(use1, m1, m0)
    idx_ref[...] = li[:, 0]
    m_ref[...] = m[:, 0]


def _k1(z16, cn16):
    return pl.pallas_call(
        _k1_body,
        grid=(NI,),
        in_specs=[
            pl.BlockSpec((BM, E), lambda i: (i, 0)),
            pl.BlockSpec((N, E), lambda i: (0, 0)),
        ],
        out_specs=[
            pl.BlockSpec((BM,), lambda i: (i,)),
            pl.BlockSpec((BM,), lambda i: (i,)),
        ],
        out_shape=[
            jax.ShapeDtypeStruct((N,), jnp.int32),
            jax.ShapeDtypeStruct((N,), jnp.float32),
        ],
    )(z16, cn16)


def _sca(idx, nc, zeros):
    mesh = plsc.VectorSubcoreMesh(core_axis_name="c", subcore_axis_name="s")
    cp = pltpu.CompilerParams()
    if "needs_layout_passes" in pltpu.CompilerParams.__dataclass_fields__:
        cp = dataclasses.replace(cp, needs_layout_passes=False)

    @functools.partial(
        pl.kernel,
        compiler_params=cp,
        out_type=[
            jax.ShapeDtypeStruct((N,), jnp.float32),
            jax.ShapeDtypeStruct((2, N), jnp.float32),
        ],
        mesh=mesh,
        scratch_types=[
            pltpu.VMEM((RPW,), jnp.int32),
            pltpu.VMEM((NCH, CH), jnp.int32),
            pltpu.VMEM((N,), jnp.float32),
            pltpu.VMEM((RPW,), jnp.float32),
            pltpu.VMEM((CH,), jnp.float32),
            pltpu.VMEM_SHARED((N,), jnp.float32),
        ],
    )
    def k(idx_hbm, nc_hbm, z_hbm, ncg_hbm, cnt_hbm,
          idx_v, idx2_v, nc_v, ncg_v, ones_v, hist_sh):
        cid = lax.axis_index("c")
        sid = lax.axis_index("s")
        wid = sid * 2 + cid
        base = wid * RPW
        pltpu.sync_copy(idx_hbm.at[pl.ds(base, RPW)], idx_v)
        for kk in range(NCH):
            pltpu.sync_copy(idx_hbm.at[pl.ds(base + kk * CH, CH)],
                            idx2_v.at[kk])
        pltpu.sync_copy(nc_hbm, nc_v)
        for t in range(CH // L):
            ones_v[pl.ds(t * L, L)] = jnp.full((L,), 1.0, jnp.float32)

        @pl.when(sid == 0)
        def _():
            pltpu.sync_copy(z_hbm, hist_sh)

        plsc.subcore_barrier()

        for kk in range(NCH):
            pltpu.sync_copy(ones_v, hist_sh.at[idx2_v.at[kk]], add=True)
        plsc.subcore_barrier()

        @pl.when(sid == 0)
        def _():
            pltpu.sync_copy(hist_sh, cnt_hbm.at[cid])

        for t in range(RPW // L):
            ig = idx_v[pl.ds(t * L, L)]
            ncg_v[pl.ds(t * L, L)] = plsc.load_gather(nc_v, [ig])
        pltpu.sync_copy(ncg_v, ncg_hbm.at[pl.ds(base, RPW)])

    return k(idx, nc, zeros)


def _k2_body(ca_ref, cb_ref, m_ref, nz_ref, ncg_ref, dest_ref, label_ref,
             loss_ref):
    used = jnp.minimum(ca_ref[...] + cb_ref[...], 1.0)
    usedT = used.T
    rio = lax.broadcasted_iota(jnp.int32, (128, 1), 0)
    cio = lax.broadcasted_iota(jnp.int32, (1, 64), 1)
    r128 = lax.broadcasted_iota(jnp.int32, (128, 128), 0)
    c128 = lax.broadcasted_iota(jnp.int32, (128, 128), 1)
    lower_incl = (c128 <= r128).astype(jnp.float32)
    inclT = lax.dot_general(lower_incl, usedT, (((1,), (0,)), ((), ())),
                            preferred_element_type=jnp.float32)
    coltot = inclT[127:128, :]
    a64 = lax.broadcasted_iota(jnp.int32, (64, 64), 0)
    b64 = lax.broadcasted_iota(jnp.int32, (64, 64), 1)
    upper_strict = (a64 < b64).astype(jnp.float32)
    coloff = lax.dot_general(coltot, upper_strict, (((1,), (0,)), ((), ())),
                             preferred_element_type=jnp.float32)
    csumT = inclT + coloff
    nu = jnp.max(csumT)
    flatj = (rio + 128 * cio).astype(jnp.float32)
    destT = jnp.where(usedT > 0.5, csumT - 1.0,
                      nu + flatj - csumT).astype(jnp.int32)
    labelT = (flatj < nu).astype(jnp.float32)
    dest_ref[...] = destT.T
    label_ref[...] = labelT.T
    mm = m_ref[...]
    nz = nz_ref[...]
    ncg = ncg_ref[...]
    tot = jnp.sum(nz * nz - 2.0 * mm * nz * ncg + ncg * ncg)
    loss_ref[...] = jnp.full((8, 128), tot * ((1.0 + BETA) / (N * E)),
                             jnp.float32)


def _k2(ca, cb, m8, nz8, ncg8):
    return pl.pallas_call(
        _k2_body,
        in_specs=[
            pl.BlockSpec((64, 128), lambda: (0, 0)),
            pl.BlockSpec((64, 128), lambda: (0, 0)),
            pl.BlockSpec((8, N // 8), lambda: (0, 0)),
            pl.BlockSpec((8, N // 8), lambda: (0, 0)),
            pl.BlockSpec((8, N // 8), lambda: (0, 0)),
        ],
        out_specs=[
            pl.BlockSpec((64, 128), lambda: (0, 0)),
            pl.BlockSpec((64, 128), lambda: (0, 0)),
            pl.BlockSpec((8, 128), lambda: (0, 0)),
        ],
        out_shape=[
            jax.ShapeDtypeStruct((64, 128), jnp.int32),
            jax.ShapeDtypeStruct((64, 128), jnp.float32),
            jax.ShapeDtypeStruct((8, 128), jnp.float32),
        ],
    )(ca, cb, m8, nz8, ncg8)


def _k3(codebook, dest3):
    mesh = plsc.VectorSubcoreMesh(core_axis_name="c", subcore_axis_name="s")

    @functools.partial(
        pl.kernel,
        out_type=jax.ShapeDtypeStruct((N, E), jnp.float32),
        mesh=mesh,
        scratch_types=[
            pltpu.VMEM((NCH, CH), jnp.int32),
            pltpu.VMEM((CH, E), jnp.float32),
        ],
    )
    def k(cb_hbm, d_hbm, out_hbm, idx_v, rows_v):
        cid = lax.axis_index("c")
        sid = lax.axis_index("s")
        wid = sid * 2 + cid
        pltpu.sync_copy(d_hbm.at[wid], idx_v)
        for kk in range(NCH):
            base = wid * RPW + kk * CH
            pltpu.sync_copy(cb_hbm.at[pl.ds(base, CH)], rows_v)
            pltpu.sync_copy(rows_v, out_hbm.at[idx_v.at[kk]])

    return k(codebook, dest3)


def kernel(z, codebook):
    zn = _normalize_rows(z)
    cn = _normalize_rows(codebook)
    z16, cn16 = lax.optimization_barrier(
        (zn.astype(jnp.bfloat16), cn.astype(jnp.bfloat16)))
    nz = jnp.linalg.norm(z, ord=2, axis=-1)
    nc = jnp.linalg.norm(codebook, ord=2, axis=-1)
    idx, m = _k1(z16, cn16)
    ncg, cnt = _sca(idx, nc, jnp.zeros((N,), jnp.float32))
    dest, label, lossbuf = _k2(cnt[0].reshape(64, 128),
                               cnt[1].reshape(64, 128),
                               m.reshape(8, N // 8),
                               nz.reshape(8, N // 8),
                               ncg.reshape(8, N // 8))
    emb = _k3(codebook, dest.reshape(NW, NCH, CH))
    return emb, label.reshape(-1), lossbuf[0, 0]
